# baseline (speedup 1.0000x reference)
import jax
import jax.numpy as jnp
import numpy as np
from jax import lax
from jax.experimental import pallas as pl
from jax.experimental.pallas import tpu as pltpu
from jax.experimental.pallas import tpu_sc as plsc

N = 32768
NSEG = 128
NTILES = 16
CHUNK = N // NTILES
VECS = CHUNK // 16
UNROLL = 8
NSTAT = 6
NGRP = NSEG // 16
LSTRIDE = NSEG + 1
ACC = 16 * LSTRIDE

_ZEROS_NP = np.zeros((NSTAT * ACC,), np.float32)


def _tree_sum(vs):
    vs = list(vs)
    while len(vs) > 1:
        nxt = [vs[i] + vs[i + 1] for i in range(0, len(vs) - 1, 2)]
        if len(vs) % 2:
            nxt.append(vs[-1])
        vs = nxt
    return vs[0]


def _newton_sqrt(d):
    i = plsc.bitcast(d, jnp.int32)
    i = (i >> 1) + jnp.full((16,), 0x1FBD1DF6, jnp.int32)
    y = plsc.bitcast(i, jnp.float32)
    half = jnp.full((16,), 0.5, jnp.float32)
    for _ in range(3):
        y = half * (y + d / y)
    return y


def _body(true_hbm, pred_hbm, loc_hbm, zeros_hbm, out_hbm,
          t_v, p_v, s_v, acc, part, red, out_v, shared,
          sem0, sem1, sem2, sem3):
    wid = lax.axis_index("s")
    base = wid * CHUNK
    c0 = pltpu.async_copy(true_hbm.at[pl.ds(base, CHUNK)], t_v, sem0)
    c1 = pltpu.async_copy(pred_hbm.at[pl.ds(base, CHUNK)], p_v, sem1)
    c2 = pltpu.async_copy(loc_hbm.at[pl.ds(base, CHUNK)], s_v, sem2)
    c3 = pltpu.async_copy(zeros_hbm, acc, sem3)
    c0.wait(); c1.wait(); c2.wait(); c3.wait()

    zeros = jnp.zeros((16,), jnp.float32)
    ones = jnp.full((16,), 1.0, jnp.float32)
    lane = lax.iota(jnp.int32, 16) * LSTRIDE

    def scat_body(i, _):
        for u in range(UNROLL):
            b = (i * UNROLL + u) * 16
            seg = s_v[pl.ds(b, 16)]
            t = t_v[pl.ds(b, 16)]
            p = p_v[pl.ds(b, 16)]
            idx = lane + seg
            plsc.addupdate_scatter(acc, [idx], ones)
            plsc.addupdate_scatter(acc, [idx + ACC], t)
            plsc.addupdate_scatter(acc, [idx + 2 * ACC], p)
            plsc.addupdate_scatter(acc, [idx + 3 * ACC], t * t)
            plsc.addupdate_scatter(acc, [idx + 4 * ACC], p * p)
            plsc.addupdate_scatter(acc, [idx + 5 * ACC], t * p)
        return 0
    lax.fori_loop(0, VECS // UNROLL, scat_body, 0)

    for k in range(NSTAT):
        for g in range(NGRP):
            v = _tree_sum(acc[pl.ds(k * ACC + l * LSTRIDE + g * 16, 16)]
                          for l in range(16))
            part[pl.ds(k * NSEG + g * 16, 16)] = v

    pltpu.sync_copy(part, shared.at[wid])
    plsc.subcore_barrier()

    @pl.when(wid == 0)
    def _():
        pltpu.sync_copy(shared, red)

        total = zeros
        for g in range(NGRP):
            stats = []
            for k in range(NSTAT):
                stats.append(_tree_sum(
                    red[w, pl.ds(k * NSEG + g * 16, 16)]
                    for w in range(NTILES)))
            cnt, st, sp, stt, spp, stp = stats
            present = cnt > zeros
            n = jnp.maximum(cnt, ones)
            sxy = stp - st * sp / n
            sx2 = jnp.maximum(stt - st * st / n, zeros)
            sy2 = jnp.maximum(spp - sp * sp / n, zeros)
            mse = (stt - 2.0 * stp + spp) / n
            d = jnp.where(present, sx2 * sy2, ones)
            r = _newton_sqrt(d)
            pcc = sxy / (r + jnp.full((16,), 1e-7, jnp.float32))
            total = total + jnp.where(present, mse * (ones - pcc), zeros)

        s = jnp.sum(total)
        out_v[...] = ones * s
        pltpu.sync_copy(out_v, out_hbm)


@jax.jit
def _launch(true, predicted, loc_no):
    mesh = plsc.VectorSubcoreMesh(
        core_axis_name="c", subcore_axis_name="s", num_cores=1)
    k = pl.kernel(
        _body,
        out_type=jax.ShapeDtypeStruct((16,), jnp.float32),
        mesh=mesh,
        compiler_params=pltpu.CompilerParams(needs_layout_passes=False),
        scratch_types=[
            pltpu.VMEM((CHUNK,), jnp.float32),
            pltpu.VMEM((CHUNK,), jnp.float32),
            pltpu.VMEM((CHUNK,), jnp.int32),
            pltpu.VMEM((NSTAT * ACC,), jnp.float32),
            pltpu.VMEM((NSTAT * NSEG,), jnp.float32),
            pltpu.VMEM((NTILES, NSTAT * NSEG), jnp.float32),
            pltpu.VMEM((16,), jnp.float32),
            pltpu.VMEM_SHARED((NTILES, NSTAT * NSEG), jnp.float32),
            pltpu.SemaphoreType.DMA,
            pltpu.SemaphoreType.DMA,
            pltpu.SemaphoreType.DMA,
            pltpu.SemaphoreType.DMA,
        ],
    )
    return k(true, predicted, loc_no, jnp.asarray(_ZEROS_NP))


def kernel(true, predicted, loc_no):
    return _launch(true, predicted, loc_no)[0]


# speedup vs baseline: 1.1573x; 1.1573x over previous
import jax
import jax.numpy as jnp
import numpy as np
from jax import lax
from jax.experimental import pallas as pl
from jax.experimental.pallas import tpu as pltpu
from jax.experimental.pallas import tpu_sc as plsc

N = 32768
NSEG = 128
NTILES = 16
CHUNK = N // NTILES
VECS = CHUNK // 16
UNROLL = 4
NSTAT = 6
NGRP = NSEG // 16
LSTRIDE = NSEG + 1
ACC = 16 * LSTRIDE

_ZEROS_NP = np.zeros((NSTAT * ACC,), np.float32)


def _tree_sum(vs):
    vs = list(vs)
    while len(vs) > 1:
        nxt = [vs[i] + vs[i + 1] for i in range(0, len(vs) - 1, 2)]
        if len(vs) % 2:
            nxt.append(vs[-1])
        vs = nxt
    return vs[0]


def _newton_sqrt(d):
    i = plsc.bitcast(d, jnp.int32)
    i = (i >> 1) + jnp.full((16,), 0x1FBD1DF6, jnp.int32)
    y = plsc.bitcast(i, jnp.float32)
    half = jnp.full((16,), 0.5, jnp.float32)
    for _ in range(3):
        y = half * (y + d / y)
    return y


def _body(true_hbm, pred_hbm, loc_hbm, zeros_hbm, out_hbm,
          t_v, p_v, s_v, acc, part, red, out_v, shared,
          sem0, sem1, sem2, sem3):
    wid = lax.axis_index("s")
    base = wid * CHUNK
    c0 = pltpu.async_copy(true_hbm.at[pl.ds(base, CHUNK)], t_v, sem0)
    c1 = pltpu.async_copy(pred_hbm.at[pl.ds(base, CHUNK)], p_v, sem1)
    c2 = pltpu.async_copy(loc_hbm.at[pl.ds(base, CHUNK)], s_v, sem2)
    c3 = pltpu.async_copy(zeros_hbm, acc, sem3)
    c0.wait(); c1.wait(); c2.wait(); c3.wait()

    zeros = jnp.zeros((16,), jnp.float32)
    ones = jnp.full((16,), 1.0, jnp.float32)
    lane = lax.iota(jnp.int32, 16) * LSTRIDE

    def scat_body(i, _):
        for u in range(UNROLL):
            b = (i * UNROLL + u) * 16
            seg = s_v[pl.ds(b, 16)]
            t = t_v[pl.ds(b, 16)]
            p = p_v[pl.ds(b, 16)]
            idx = lane + seg
            plsc.addupdate_scatter(acc, [idx], ones)
            plsc.addupdate_scatter(acc, [idx + ACC], t)
            plsc.addupdate_scatter(acc, [idx + 2 * ACC], p)
            plsc.addupdate_scatter(acc, [idx + 3 * ACC], t * t)
            plsc.addupdate_scatter(acc, [idx + 4 * ACC], p * p)
            plsc.addupdate_scatter(acc, [idx + 5 * ACC], t * p)
        return 0
    lax.fori_loop(0, VECS // UNROLL, scat_body, 0)

    for k in range(NSTAT):
        def grp_body(g, _, k=k):
            v = _tree_sum(acc[pl.ds(k * ACC + l * LSTRIDE + g * 16, 16)]
                          for l in range(16))
            part[pl.ds(k * NSEG + g * 16, 16)] = v
            return 0
        lax.fori_loop(0, NGRP, grp_body, 0)

    pltpu.sync_copy(part, shared.at[wid])
    plsc.subcore_barrier()

    @pl.when(wid == 0)
    def _():
        pltpu.sync_copy(shared, red)

        total = zeros
        for g in range(NGRP):
            def tile_body(w, carry, g=g):
                return tuple(
                    carry[k] + red[w, pl.ds(k * NSEG + g * 16, 16)]
                    for k in range(NSTAT))
            stats = lax.fori_loop(0, NTILES, tile_body, (zeros,) * NSTAT)
            cnt, st, sp, stt, spp, stp = stats
            present = cnt > zeros
            n = jnp.maximum(cnt, ones)
            sxy = stp - st * sp / n
            sx2 = jnp.maximum(stt - st * st / n, zeros)
            sy2 = jnp.maximum(spp - sp * sp / n, zeros)
            mse = (stt - 2.0 * stp + spp) / n
            d = jnp.where(present, sx2 * sy2, ones)
            r = _newton_sqrt(d)
            pcc = sxy / (r + jnp.full((16,), 1e-7, jnp.float32))
            total = total + jnp.where(present, mse * (ones - pcc), zeros)

        s = jnp.sum(total)
        out_v[...] = ones * s
        pltpu.sync_copy(out_v, out_hbm)


@jax.jit
def _launch(true, predicted, loc_no):
    mesh = plsc.VectorSubcoreMesh(
        core_axis_name="c", subcore_axis_name="s", num_cores=1)
    k = pl.kernel(
        _body,
        out_type=jax.ShapeDtypeStruct((16,), jnp.float32),
        mesh=mesh,
        compiler_params=pltpu.CompilerParams(needs_layout_passes=False),
        scratch_types=[
            pltpu.VMEM((CHUNK,), jnp.float32),
            pltpu.VMEM((CHUNK,), jnp.float32),
            pltpu.VMEM((CHUNK,), jnp.int32),
            pltpu.VMEM((NSTAT * ACC,), jnp.float32),
            pltpu.VMEM((NSTAT * NSEG,), jnp.float32),
            pltpu.VMEM((NTILES, NSTAT * NSEG), jnp.float32),
            pltpu.VMEM((16,), jnp.float32),
            pltpu.VMEM_SHARED((NTILES, NSTAT * NSEG), jnp.float32),
            pltpu.SemaphoreType.DMA,
            pltpu.SemaphoreType.DMA,
            pltpu.SemaphoreType.DMA,
            pltpu.SemaphoreType.DMA,
        ],
    )
    return k(true, predicted, loc_no, jnp.asarray(_ZEROS_NP))


def kernel(true, predicted, loc_no):
    return _launch(true, predicted, loc_no)[0]


# speedup vs baseline: 1.1901x; 1.0283x over previous
import jax
import jax.numpy as jnp
import numpy as np
from jax import lax
from jax.experimental import pallas as pl
from jax.experimental.pallas import tpu as pltpu
from jax.experimental.pallas import tpu_sc as plsc

N = 32768
NSEG = 128
NTILES = 16
CHUNK = N // NTILES
VECS = CHUNK // 16
UNROLL = 4
NSTAT = 6
NGRP = NSEG // 16
LSTRIDE = NSEG + 1
ACC = 16 * LSTRIDE

_ZEROS_NP = np.zeros((NSTAT * ACC,), np.float32)


def _tree_sum(vs):
    vs = list(vs)
    while len(vs) > 1:
        nxt = [vs[i] + vs[i + 1] for i in range(0, len(vs) - 1, 2)]
        if len(vs) % 2:
            nxt.append(vs[-1])
        vs = nxt
    return vs[0]


def _newton_sqrt(d):
    i = plsc.bitcast(d, jnp.int32)
    i = (i >> 1) + jnp.full((16,), 0x1FBD1DF6, jnp.int32)
    y = plsc.bitcast(i, jnp.float32)
    half = jnp.full((16,), 0.5, jnp.float32)
    for _ in range(3):
        y = half * (y + d / y)
    return y


def _body(true_hbm, pred_hbm, loc_hbm, zeros_hbm, out_hbm,
          t_v, p_v, s_v, acc, part, red, out_v, shared,
          sem0, sem1, sem2, sem3):
    wid = lax.axis_index("s")
    base = wid * CHUNK
    c0 = pltpu.async_copy(true_hbm.at[pl.ds(base, CHUNK)], t_v, sem0)
    c1 = pltpu.async_copy(pred_hbm.at[pl.ds(base, CHUNK)], p_v, sem1)
    c2 = pltpu.async_copy(loc_hbm.at[pl.ds(base, CHUNK)], s_v, sem2)
    c3 = pltpu.async_copy(zeros_hbm, acc, sem3)
    c0.wait(); c1.wait(); c2.wait(); c3.wait()

    zeros = jnp.zeros((16,), jnp.float32)
    ones = jnp.full((16,), 1.0, jnp.float32)
    lane = lax.iota(jnp.int32, 16) * LSTRIDE

    def scat_body(i, _):
        for u in range(UNROLL):
            b = (i * UNROLL + u) * 16
            seg = s_v[pl.ds(b, 16)]
            t = t_v[pl.ds(b, 16)]
            p = p_v[pl.ds(b, 16)]
            idx = lane + seg
            plsc.addupdate_scatter(acc, [idx], ones)
            plsc.addupdate_scatter(acc, [idx + ACC], t)
            plsc.addupdate_scatter(acc, [idx + 2 * ACC], p)
            plsc.addupdate_scatter(acc, [idx + 3 * ACC], t * t)
            plsc.addupdate_scatter(acc, [idx + 4 * ACC], p * p)
            plsc.addupdate_scatter(acc, [idx + 5 * ACC], t * p)
        return 0
    lax.fori_loop(0, VECS // UNROLL, scat_body, 0)

    for k in range(NSTAT):
        def grp_body(g, _, k=k):
            v = _tree_sum(acc[pl.ds(k * ACC + l * LSTRIDE + g * 16, 16)]
                          for l in range(16))
            part[pl.ds(k * NSEG + g * 16, 16)] = v
            return 0
        lax.fori_loop(0, NGRP, grp_body, 0)

    pltpu.sync_copy(part, shared.at[wid])
    plsc.subcore_barrier()

    @pl.when(wid == 0)
    def _():
        pltpu.sync_copy(shared, red)

        def grp_combine(g, total):
            def tile_body(w, carry):
                return tuple(
                    carry[k] + red[w, pl.ds(k * NSEG + g * 16, 16)]
                    for k in range(NSTAT))
            stats = lax.fori_loop(0, NTILES, tile_body, (zeros,) * NSTAT)
            cnt, st, sp, stt, spp, stp = stats
            present = cnt > zeros
            n = jnp.maximum(cnt, ones)
            sxy = stp - st * sp / n
            sx2 = jnp.maximum(stt - st * st / n, zeros)
            sy2 = jnp.maximum(spp - sp * sp / n, zeros)
            mse = (stt - 2.0 * stp + spp) / n
            d = jnp.where(present, sx2 * sy2, ones)
            r = _newton_sqrt(d)
            pcc = sxy / (r + jnp.full((16,), 1e-7, jnp.float32))
            return total + jnp.where(present, mse * (ones - pcc), zeros)

        total = lax.fori_loop(0, NGRP, grp_combine, zeros)
        s = jnp.sum(total)
        out_v[...] = ones * s
        pltpu.sync_copy(out_v, out_hbm)


@jax.jit
def _launch(true, predicted, loc_no):
    mesh = plsc.VectorSubcoreMesh(
        core_axis_name="c", subcore_axis_name="s", num_cores=1)
    k = pl.kernel(
        _body,
        out_type=jax.ShapeDtypeStruct((16,), jnp.float32),
        mesh=mesh,
        compiler_params=pltpu.CompilerParams(needs_layout_passes=False),
        scratch_types=[
            pltpu.VMEM((CHUNK,), jnp.float32),
            pltpu.VMEM((CHUNK,), jnp.float32),
            pltpu.VMEM((CHUNK,), jnp.int32),
            pltpu.VMEM((NSTAT * ACC,), jnp.float32),
            pltpu.VMEM((NSTAT * NSEG,), jnp.float32),
            pltpu.VMEM((NTILES, NSTAT * NSEG), jnp.float32),
            pltpu.VMEM((16,), jnp.float32),
            pltpu.VMEM_SHARED((NTILES, NSTAT * NSEG), jnp.float32),
            pltpu.SemaphoreType.DMA,
            pltpu.SemaphoreType.DMA,
            pltpu.SemaphoreType.DMA,
            pltpu.SemaphoreType.DMA,
        ],
    )
    return k(true, predicted, loc_no, jnp.asarray(_ZEROS_NP))


def kernel(true, predicted, loc_no):
    return _launch(true, predicted, loc_no)[0]


# speedup vs baseline: 1.1952x; 1.0043x over previous
import jax
import jax.numpy as jnp
import numpy as np
from jax import lax
from jax.experimental import pallas as pl
from jax.experimental.pallas import tpu as pltpu
from jax.experimental.pallas import tpu_sc as plsc

N = 32768
NSEG = 128
NTILES = 16
CHUNK = N // NTILES
VECS = CHUNK // 16
UNROLL = 4
NSTAT = 6
NGRP = NSEG // 16
LSTRIDE = NSEG + 1
ACC = 16 * LSTRIDE

_ZEROS_NP = np.zeros((NSTAT * ACC,), np.float32)


def _tree_sum(vs):
    vs = list(vs)
    while len(vs) > 1:
        nxt = [vs[i] + vs[i + 1] for i in range(0, len(vs) - 1, 2)]
        if len(vs) % 2:
            nxt.append(vs[-1])
        vs = nxt
    return vs[0]


def _newton_sqrt(d):
    i = plsc.bitcast(d, jnp.int32)
    i = (i >> 1) + jnp.full((16,), 0x1FBD1DF6, jnp.int32)
    y = plsc.bitcast(i, jnp.float32)
    half = jnp.full((16,), 0.5, jnp.float32)
    for _ in range(3):
        y = half * (y + d / y)
    return y


def _body(true_hbm, pred_hbm, loc_hbm, zeros_hbm, out_hbm,
          t_v, p_v, s_v, acc, part, red, out_v, shared,
          sem0, sem1, sem2, sem3):
    wid = lax.axis_index("s")
    base = wid * CHUNK
    c0 = pltpu.async_copy(true_hbm.at[pl.ds(base, CHUNK)], t_v, sem0)
    c1 = pltpu.async_copy(pred_hbm.at[pl.ds(base, CHUNK)], p_v, sem1)
    c2 = pltpu.async_copy(loc_hbm.at[pl.ds(base, CHUNK)], s_v, sem2)
    c3 = pltpu.async_copy(zeros_hbm, acc, sem3)
    c0.wait(); c1.wait(); c2.wait(); c3.wait()

    zeros = jnp.zeros((16,), jnp.float32)
    ones = jnp.full((16,), 1.0, jnp.float32)
    lane = lax.iota(jnp.int32, 16) * LSTRIDE

    def scat_body(i, _):
        for u in range(UNROLL):
            b = (i * UNROLL + u) * 16
            seg = s_v[pl.ds(b, 16)]
            t = t_v[pl.ds(b, 16)]
            p = p_v[pl.ds(b, 16)]
            idx = lane + seg
            plsc.addupdate_scatter(acc, [idx], ones)
            plsc.addupdate_scatter(acc, [idx + ACC], t)
            plsc.addupdate_scatter(acc, [idx + 2 * ACC], p)
            plsc.addupdate_scatter(acc, [idx + 3 * ACC], t * t)
            plsc.addupdate_scatter(acc, [idx + 4 * ACC], p * p)
            plsc.addupdate_scatter(acc, [idx + 5 * ACC], t * p)
        return 0
    lax.fori_loop(0, VECS // UNROLL, scat_body, 0)

    def fold_body(pair, _):
        k = pair // NGRP
        g = pair % NGRP
        v = _tree_sum(acc[pl.ds(k * ACC + l * LSTRIDE + g * 16, 16)]
                      for l in range(16))
        part[pl.ds(k * NSEG + g * 16, 16)] = v
        return 0
    lax.fori_loop(0, NSTAT * NGRP, fold_body, 0)

    pltpu.sync_copy(part, shared.at[wid])
    plsc.subcore_barrier()

    @pl.when(wid == 0)
    def _():
        pltpu.sync_copy(shared, red)

        def grp_combine(g, total):
            def tile_body(w, carry):
                return tuple(
                    carry[k] + red[w, pl.ds(k * NSEG + g * 16, 16)]
                    for k in range(NSTAT))
            stats = lax.fori_loop(0, NTILES, tile_body, (zeros,) * NSTAT)
            cnt, st, sp, stt, spp, stp = stats
            present = cnt > zeros
            n = jnp.maximum(cnt, ones)
            sxy = stp - st * sp / n
            sx2 = jnp.maximum(stt - st * st / n, zeros)
            sy2 = jnp.maximum(spp - sp * sp / n, zeros)
            mse = (stt - 2.0 * stp + spp) / n
            d = jnp.where(present, sx2 * sy2, ones)
            r = _newton_sqrt(d)
            pcc = sxy / (r + jnp.full((16,), 1e-7, jnp.float32))
            return total + jnp.where(present, mse * (ones - pcc), zeros)

        total = lax.fori_loop(0, NGRP, grp_combine, zeros)
        s = jnp.sum(total)
        out_v[...] = ones * s
        pltpu.sync_copy(out_v, out_hbm)


@jax.jit
def _launch(true, predicted, loc_no):
    mesh = plsc.VectorSubcoreMesh(
        core_axis_name="c", subcore_axis_name="s", num_cores=1)
    k = pl.kernel(
        _body,
        out_type=jax.ShapeDtypeStruct((16,), jnp.float32),
        mesh=mesh,
        compiler_params=pltpu.CompilerParams(needs_layout_passes=False),
        scratch_types=[
            pltpu.VMEM((CHUNK,), jnp.float32),
            pltpu.VMEM((CHUNK,), jnp.float32),
            pltpu.VMEM((CHUNK,), jnp.int32),
            pltpu.VMEM((NSTAT * ACC,), jnp.float32),
            pltpu.VMEM((NSTAT * NSEG,), jnp.float32),
            pltpu.VMEM((NTILES, NSTAT * NSEG), jnp.float32),
            pltpu.VMEM((16,), jnp.float32),
            pltpu.VMEM_SHARED((NTILES, NSTAT * NSEG), jnp.float32),
            pltpu.SemaphoreType.DMA,
            pltpu.SemaphoreType.DMA,
            pltpu.SemaphoreType.DMA,
            pltpu.SemaphoreType.DMA,
        ],
    )
    return k(true, predicted, loc_no, jnp.asarray(_ZEROS_NP))


def kernel(true, predicted, loc_no):
    return _launch(true, predicted, loc_no)[0]


# speedup vs baseline: 1.1968x; 1.0014x over previous
import jax
import jax.numpy as jnp
import numpy as np
from jax import lax
from jax.experimental import pallas as pl
from jax.experimental.pallas import tpu as pltpu
from jax.experimental.pallas import tpu_sc as plsc

N = 32768
NSEG = 128
NTILES = 16
CHUNK = N // NTILES
VECS = CHUNK // 16
UNROLL = 4
NSTAT = 6
NGRP = NSEG // 16
LSTRIDE = NSEG + 1
ACC = 16 * LSTRIDE

_ZEROS_NP = np.zeros((NSTAT * ACC,), np.float32)


def _tree_sum(vs):
    vs = list(vs)
    while len(vs) > 1:
        nxt = [vs[i] + vs[i + 1] for i in range(0, len(vs) - 1, 2)]
        if len(vs) % 2:
            nxt.append(vs[-1])
        vs = nxt
    return vs[0]


def _newton_sqrt(d):
    i = plsc.bitcast(d, jnp.int32)
    i = (i >> 1) + jnp.full((16,), 0x1FBD1DF6, jnp.int32)
    y = plsc.bitcast(i, jnp.float32)
    half = jnp.full((16,), 0.5, jnp.float32)
    for _ in range(3):
        y = half * (y + d / y)
    return y


def _body(true_hbm, pred_hbm, loc_hbm, zeros_hbm, out_hbm,
          t_v, p_v, s_v, acc, part, red8, red, out_v, shared,
          sem0, sem1, sem2, sem3):
    wid = lax.axis_index("s")
    base = wid * CHUNK
    c0 = pltpu.async_copy(true_hbm.at[pl.ds(base, CHUNK)], t_v, sem0)
    c1 = pltpu.async_copy(pred_hbm.at[pl.ds(base, CHUNK)], p_v, sem1)
    c2 = pltpu.async_copy(loc_hbm.at[pl.ds(base, CHUNK)], s_v, sem2)
    c3 = pltpu.async_copy(zeros_hbm, acc, sem3)
    c0.wait(); c1.wait(); c2.wait(); c3.wait()

    zeros = jnp.zeros((16,), jnp.float32)
    ones = jnp.full((16,), 1.0, jnp.float32)
    lane = lax.iota(jnp.int32, 16) * LSTRIDE

    def scat_body(i, _):
        for u in range(UNROLL):
            b = (i * UNROLL + u) * 16
            seg = s_v[pl.ds(b, 16)]
            t = t_v[pl.ds(b, 16)]
            p = p_v[pl.ds(b, 16)]
            idx = lane + seg
            plsc.addupdate_scatter(acc, [idx], ones)
            plsc.addupdate_scatter(acc, [idx + ACC], t)
            plsc.addupdate_scatter(acc, [idx + 2 * ACC], p)
            plsc.addupdate_scatter(acc, [idx + 3 * ACC], t * t)
            plsc.addupdate_scatter(acc, [idx + 4 * ACC], p * p)
            plsc.addupdate_scatter(acc, [idx + 5 * ACC], t * p)
        return 0
    lax.fori_loop(0, VECS // UNROLL, scat_body, 0)

    def fold_body(pair, _):
        k = pair // NGRP
        g = pair % NGRP
        v = _tree_sum(acc[pl.ds(k * ACC + l * LSTRIDE + g * 16, 16)]
                      for l in range(16))
        part[pl.ds(k * NSEG + g * 16, 16)] = v
        return 0
    lax.fori_loop(0, NSTAT * NGRP, fold_body, 0)

    pltpu.sync_copy(part, shared.at[wid])
    plsc.subcore_barrier()

    @pl.when(wid < NTILES // 2)
    def _():
        pltpu.sync_copy(shared.at[wid + NTILES // 2], red8)

        def add_body(i, _):
            part[pl.ds(i * 16, 16)] = (part[pl.ds(i * 16, 16)]
                                       + red8[pl.ds(i * 16, 16)])
            return 0
        lax.fori_loop(0, NSTAT * NSEG // 16, add_body, 0)
        pltpu.sync_copy(part, shared.at[wid])
    plsc.subcore_barrier()

    @pl.when(wid == 0)
    def _():
        pltpu.sync_copy(shared.at[pl.ds(0, NTILES // 2)], red)

        def grp_combine(g, total):
            def tile_body(w, carry):
                return tuple(
                    carry[k] + red[w, pl.ds(k * NSEG + g * 16, 16)]
                    for k in range(NSTAT))
            stats = lax.fori_loop(0, NTILES // 2, tile_body,
                                  (zeros,) * NSTAT)
            cnt, st, sp, stt, spp, stp = stats
            present = cnt > zeros
            n = jnp.maximum(cnt, ones)
            sxy = stp - st * sp / n
            sx2 = jnp.maximum(stt - st * st / n, zeros)
            sy2 = jnp.maximum(spp - sp * sp / n, zeros)
            mse = (stt - 2.0 * stp + spp) / n
            d = jnp.where(present, sx2 * sy2, ones)
            r = _newton_sqrt(d)
            pcc = sxy / (r + jnp.full((16,), 1e-7, jnp.float32))
            return total + jnp.where(present, mse * (ones - pcc), zeros)

        total = lax.fori_loop(0, NGRP, grp_combine, zeros)
        s = jnp.sum(total)
        out_v[...] = ones * s
        pltpu.sync_copy(out_v, out_hbm)


@jax.jit
def _launch(true, predicted, loc_no):
    mesh = plsc.VectorSubcoreMesh(
        core_axis_name="c", subcore_axis_name="s", num_cores=1)
    k = pl.kernel(
        _body,
        out_type=jax.ShapeDtypeStruct((16,), jnp.float32),
        mesh=mesh,
        compiler_params=pltpu.CompilerParams(needs_layout_passes=False),
        scratch_types=[
            pltpu.VMEM((CHUNK,), jnp.float32),
            pltpu.VMEM((CHUNK,), jnp.float32),
            pltpu.VMEM((CHUNK,), jnp.int32),
            pltpu.VMEM((NSTAT * ACC,), jnp.float32),
            pltpu.VMEM((NSTAT * NSEG,), jnp.float32),
            pltpu.VMEM((NSTAT * NSEG,), jnp.float32),
            pltpu.VMEM((NTILES // 2, NSTAT * NSEG), jnp.float32),
            pltpu.VMEM((16,), jnp.float32),
            pltpu.VMEM_SHARED((NTILES, NSTAT * NSEG), jnp.float32),
            pltpu.SemaphoreType.DMA,
            pltpu.SemaphoreType.DMA,
            pltpu.SemaphoreType.DMA,
            pltpu.SemaphoreType.DMA,
        ],
    )
    return k(true, predicted, loc_no, jnp.asarray(_ZEROS_NP))


def kernel(true, predicted, loc_no):
    return _launch(true, predicted, loc_no)[0]
